# baseline (speedup 1.0000x reference)
import functools
import jax
import jax.numpy as jnp
from jax import lax
from jax.experimental import pallas as pl
from jax.experimental.pallas import tpu as pltpu
from jax.experimental.pallas import tpu_sc as plsc

N = 10000
E = 320000
D = 128
DH = D // 2

NC = 2
NS = 16
NPAD = 10240
EPT = 20480
EP = NS * EPT
CH = 128
NCHUNK = EPT // CH
RPT = NPAD // NS
RCH = 128
RZ = 64

_SC_MESH = plsc.VectorSubcoreMesh(core_axis_name="c", subcore_axis_name="s")


@functools.partial(
    pl.kernel,
    out_type=jax.ShapeDtypeStruct((NC, 1, NPAD), jnp.float32),
    mesh=_SC_MESH,
    scratch_types=[
        pltpu.VMEM_SHARED((NPAD,), jnp.float32),
        pltpu.VMEM((2048,), jnp.float32),
        pltpu.VMEM((CH,), jnp.int32),
        pltpu.VMEM((CH,), jnp.float32),
        pltpu.SemaphoreType.DMA,
    ],
)
def _deg_kernel(dst_hbm, ew_hbm, out_hbm, acc, zbuf, didx, wbuf, sem):
    c = lax.axis_index("c")
    s = lax.axis_index("s")
    wid = c * NS + s

    @pl.when(s == 0)
    def _():
        def z(i, _):
            zbuf[pl.ds(i * 16, 16)] = jnp.zeros((16,), jnp.float32)
            return 0

        lax.fori_loop(0, 2048 // 16, z, 0)
        for k in range(NPAD // 2048):
            pltpu.sync_copy(zbuf, acc.at[pl.ds(k * 2048, 2048)])

    plsc.subcore_barrier()

    ebase = wid * (EP // (NC * NS))

    def chunk(j, _):
        off = ebase + j * CH
        cp1 = pltpu.async_copy(dst_hbm.at[pl.ds(off, CH)], didx, sem)
        cp2 = pltpu.async_copy(ew_hbm.at[pl.ds(off, CH)], wbuf, sem)
        cp1.wait()
        cp2.wait()
        pltpu.sync_copy(wbuf, acc.at[didx], add=True)
        return 0

    lax.fori_loop(0, EP // (NC * NS) // CH, chunk, 0)
    plsc.subcore_barrier()

    @pl.when(s == 0)
    def _():
        pltpu.sync_copy(acc, out_hbm.at[c, 0])


@functools.partial(
    pl.kernel,
    out_type=jax.ShapeDtypeStruct((NC, NPAD, DH), jnp.float32),
    mesh=_SC_MESH,
    compiler_params=pltpu.CompilerParams(use_tc_tiling_on_sc=False),
    scratch_types=[
        pltpu.VMEM_SHARED((NPAD, DH), jnp.float32),
        pltpu.VMEM_SHARED((NPAD, DH), jnp.float32),
        pltpu.VMEM((RZ, DH), jnp.float32),
        pltpu.VMEM((2, CH), jnp.int32),
        pltpu.VMEM((2, CH), jnp.int32),
        pltpu.VMEM((2, CH), jnp.float32),
        pltpu.VMEM((2, CH, DH), jnp.float32),
        pltpu.SemaphoreType.DMA,
        pltpu.SemaphoreType.DMA,
        pltpu.SemaphoreType.DMA,
        pltpu.SemaphoreType.DMA,
    ],
)
def _msg_kernel(hs_hbm, src_hbm, dst_hbm, ew_hbm, out_hbm,
                acc, hsm, zbuf, sidx, didx, ewb, rows, si0, si1, sr0, sr1):
    c = lax.axis_index("c")
    s = lax.axis_index("s")
    semi = (si0, si1)
    semr = (sr0, sr1)

    pltpu.sync_copy(hs_hbm.at[c, pl.ds(s * RPT, RPT)], hsm.at[pl.ds(s * RPT, RPT)])

    def zrow(r, _):
        for cc in range(DH // 16):
            zbuf[r, pl.ds(cc * 16, 16)] = jnp.zeros((16,), jnp.float32)
        return 0

    lax.fori_loop(0, RZ, zrow, 0)
    for k in range(RPT // RZ):
        pltpu.sync_copy(zbuf, acc.at[pl.ds(s * RPT + k * RZ, RZ)])
    plsc.subcore_barrier()

    ebase = s * EPT

    def fetch(j, b):
        off = ebase + j * CH
        pltpu.async_copy(src_hbm.at[pl.ds(off, CH)], sidx.at[b], semi[b])
        pltpu.async_copy(dst_hbm.at[pl.ds(off, CH)], didx.at[b], semi[b])
        pltpu.async_copy(ew_hbm.at[pl.ds(off, CH)], ewb.at[b], semi[b])

    def drain_idx(b):
        pltpu.make_async_copy(src_hbm.at[pl.ds(0, CH)], sidx.at[b], semi[b]).wait()
        pltpu.make_async_copy(dst_hbm.at[pl.ds(0, CH)], didx.at[b], semi[b]).wait()
        pltpu.make_async_copy(ew_hbm.at[pl.ds(0, CH)], ewb.at[b], semi[b]).wait()

    def start_gather(b):
        pltpu.async_copy(hsm.at[sidx.at[b]], rows.at[b], semr[b])

    def drain_rows(b):
        pltpu.make_async_copy(hsm.at[sidx.at[b]], rows.at[b], semr[b]).wait()

    fetch(0, 0)
    drain_idx(0)
    start_gather(0)
    fetch(1, 1)

    def body(jj, _):
        for b in (0, 1):
            j = jj * 2 + b
            drain_rows(b)

            @pl.when(j + 1 < NCHUNK)
            def _():
                drain_idx(1 - b)
                start_gather(1 - b)

            def scale(g, _):
                wv = ewb[b, pl.ds(g * 16, 16)]
                for l in range(16):
                    w = wv[l]
                    e = g * 16 + l
                    for cc in range(DH // 16):
                        rows[b, e, pl.ds(cc * 16, 16)] = (
                            rows[b, e, pl.ds(cc * 16, 16)] * w)
                return 0

            lax.fori_loop(0, CH // 16, scale, 0)
            pltpu.sync_copy(rows.at[b], acc.at[didx.at[b]], add=True)

            @pl.when(j + 2 < NCHUNK)
            def _():
                fetch(j + 2, b)
        return 0

    lax.fori_loop(0, NCHUNK // 2, body, 0)
    plsc.subcore_barrier()

    for k in range(RPT // RCH):
        r0 = s * RPT + k * RCH
        pltpu.sync_copy(acc.at[pl.ds(r0, RCH)], out_hbm.at[c, pl.ds(r0, RCH)])


_RB = 1000


def _m1_body(x_ref, w_ref, degp_ref, hs_ref, dinv_ref):
    deg = jnp.sum(degp_ref[...], axis=1) + 1.0
    dinv = lax.rsqrt(deg)
    h = lax.dot_general(x_ref[...], w_ref[...],
                        (((1,), (1,)), ((), ())),
                        preferred_element_type=jnp.float32)
    hs = h * dinv[:, None]
    hs_ref[0] = hs[:, :DH]
    hs_ref[1] = hs[:, DH:]
    dinv_ref[...] = dinv[:, None]


def _tc_stage1(x, w1, degp):
    return pl.pallas_call(
        _m1_body,
        grid=(N // _RB,),
        in_specs=[
            pl.BlockSpec((_RB, D), lambda i: (i, 0)),
            pl.BlockSpec((D, D), lambda i: (0, 0)),
            pl.BlockSpec((_RB, NC), lambda i: (i, 0)),
        ],
        out_specs=[
            pl.BlockSpec((NC, _RB, DH), lambda i: (0, i, 0)),
            pl.BlockSpec((_RB, 1), lambda i: (i, 0)),
        ],
        out_shape=[
            jax.ShapeDtypeStruct((NC, NPAD, DH), jnp.float32),
            jax.ShapeDtypeStruct((N, 1), jnp.float32),
        ],
    )(x, w1, degp)


def _m2_body(accp_ref, hsp_ref, dinv_ref, b_ref, w_ref, out_ref):
    dinv = dinv_ref[...]
    za = jnp.concatenate([accp_ref[0], accp_ref[1]], axis=1)
    zh = jnp.concatenate([hsp_ref[0], hsp_ref[1]], axis=1)
    z = dinv * (za + zh) + b_ref[...]
    r = jnp.maximum(z, 0.0)
    h2 = lax.dot_general(r, w_ref[...],
                         (((1,), (1,)), ((), ())),
                         preferred_element_type=jnp.float32)
    hs2 = h2 * dinv
    out_ref[0] = hs2[:, :DH]
    out_ref[1] = hs2[:, DH:]


def _tc_stage2(accp, hsp, dinv, b1, w2):
    return pl.pallas_call(
        _m2_body,
        grid=(N // _RB,),
        in_specs=[
            pl.BlockSpec((NC, _RB, DH), lambda i: (0, i, 0)),
            pl.BlockSpec((NC, _RB, DH), lambda i: (0, i, 0)),
            pl.BlockSpec((_RB, 1), lambda i: (i, 0)),
            pl.BlockSpec((1, D), lambda i: (0, 0)),
            pl.BlockSpec((D, D), lambda i: (0, 0)),
        ],
        out_specs=pl.BlockSpec((NC, _RB, DH), lambda i: (0, i, 0)),
        out_shape=jax.ShapeDtypeStruct((NC, NPAD, DH), jnp.float32),
    )(accp, hsp, dinv, b1, w2)


def _m3_body(accp_ref, hsp_ref, dinv_ref, b_ref, out_ref):
    dinv = dinv_ref[...]
    za = jnp.concatenate([accp_ref[0], accp_ref[1]], axis=1)
    zh = jnp.concatenate([hsp_ref[0], hsp_ref[1]], axis=1)
    out_ref[...] = dinv * (za + zh) + b_ref[...]


def _tc_stage3(accp, hsp, dinv, b2):
    return pl.pallas_call(
        _m3_body,
        grid=(N // _RB,),
        in_specs=[
            pl.BlockSpec((NC, _RB, DH), lambda i: (0, i, 0)),
            pl.BlockSpec((NC, _RB, DH), lambda i: (0, i, 0)),
            pl.BlockSpec((_RB, 1), lambda i: (i, 0)),
            pl.BlockSpec((1, D), lambda i: (0, 0)),
        ],
        out_specs=pl.BlockSpec((_RB, D), lambda i: (i, 0)),
        out_shape=jax.ShapeDtypeStruct((N, D), jnp.float32),
    )(accp, hsp, dinv, b2)


def kernel(x, edge_index, edge_weight, W1, b1, W2, b2):
    src = edge_index[0]
    dst = edge_index[1]
    pad = EP - E
    srcp = jnp.concatenate([src, jnp.zeros((pad,), src.dtype)])
    dstp = jnp.concatenate([dst, jnp.zeros((pad,), dst.dtype)])
    ewp = jnp.concatenate([edge_weight, jnp.zeros((pad,), edge_weight.dtype)])
    b1r = b1.reshape(1, D)
    b2r = b2.reshape(1, D)

    degp = _deg_kernel(dstp, ewp)
    degt = degp.reshape(NC, NPAD).T
    hs1, dinv = _tc_stage1(x, W1, degt)
    acc1 = _msg_kernel(hs1, srcp, dstp, ewp)
    hs2 = _tc_stage2(acc1, hs1, dinv, b1r, W2)
    acc2 = _msg_kernel(hs2, srcp, dstp, ewp)
    out = _tc_stage3(acc2, hs2, dinv, b2r)
    return out


# speedup vs baseline: 1.5685x; 1.5685x over previous
import functools
import jax
import jax.numpy as jnp
from jax import lax
from jax.experimental import pallas as pl
from jax.experimental.pallas import tpu as pltpu
from jax.experimental.pallas import tpu_sc as plsc

N = 10000
E = 320000
D = 128
DH = D // 2

NC = 2
NS = 16
NPAD = 10240
EPT = 20480
EP = NS * EPT
CH = 256
Q = CH // 128
NCHUNK = EPT // CH
RPT = NPAD // NS
RCH = 128
RZ = 64

_SC_MESH = plsc.VectorSubcoreMesh(core_axis_name="c", subcore_axis_name="s")


@functools.partial(
    pl.kernel,
    out_type=jax.ShapeDtypeStruct((NC, 1, NPAD), jnp.float32),
    mesh=_SC_MESH,
    scratch_types=[
        pltpu.VMEM_SHARED((NPAD,), jnp.float32),
        pltpu.VMEM((2048,), jnp.float32),
        pltpu.VMEM((CH,), jnp.int32),
        pltpu.VMEM((CH,), jnp.float32),
        pltpu.SemaphoreType.DMA,
    ],
)
def _deg_kernel(dst_hbm, ew_hbm, out_hbm, acc, zbuf, didx, wbuf, sem):
    c = lax.axis_index("c")
    s = lax.axis_index("s")
    wid = c * NS + s

    @pl.when(s == 0)
    def _():
        def z(i, _):
            zbuf[pl.ds(i * 16, 16)] = jnp.zeros((16,), jnp.float32)
            return 0

        lax.fori_loop(0, 2048 // 16, z, 0)
        for k in range(NPAD // 2048):
            pltpu.sync_copy(zbuf, acc.at[pl.ds(k * 2048, 2048)])

    plsc.subcore_barrier()

    ebase = wid * (EP // (NC * NS))

    def chunk(j, _):
        off = ebase + j * CH
        cp1 = pltpu.async_copy(dst_hbm.at[pl.ds(off, CH)], didx, sem)
        cp2 = pltpu.async_copy(ew_hbm.at[pl.ds(off, CH)], wbuf, sem)
        cp1.wait()
        cp2.wait()
        pltpu.sync_copy(wbuf, acc.at[didx], add=True)
        return 0

    lax.fori_loop(0, EP // (NC * NS) // CH, chunk, 0)
    plsc.subcore_barrier()

    @pl.when(s == 0)
    def _():
        pltpu.sync_copy(acc, out_hbm.at[c, 0])


@functools.partial(
    pl.kernel,
    out_type=jax.ShapeDtypeStruct((NC, NPAD, DH), jnp.float32),
    mesh=_SC_MESH,
    compiler_params=pltpu.CompilerParams(use_tc_tiling_on_sc=False),
    scratch_types=[
        pltpu.VMEM_SHARED((NPAD, DH), jnp.float32),
        pltpu.VMEM_SHARED((NPAD, DH), jnp.float32),
        pltpu.VMEM((RZ, DH), jnp.float32),
        pltpu.VMEM((2, Q, 128), jnp.int32),
        pltpu.VMEM((2, Q, 128), jnp.int32),
        pltpu.VMEM((2, Q, 128), jnp.int32),
        pltpu.VMEM((2, Q, 128), jnp.float32),
        pltpu.VMEM((2, CH, DH), jnp.float32),
        pltpu.SemaphoreType.DMA,
        pltpu.SemaphoreType.DMA,
        pltpu.SemaphoreType.DMA,
        pltpu.SemaphoreType.DMA,
        pltpu.SemaphoreType.DMA,
        pltpu.SemaphoreType.DMA,
    ],
)
def _msg_kernel(hs_hbm, src_hbm, dst_hbm, ew_hbm, out_hbm,
                acc, hsm, zbuf, sidx, didx, didx_s, ewb, rows,
                si0, si1, sr0, sr1, sw0, sw1):
    c = lax.axis_index("c")
    s = lax.axis_index("s")
    semi = (si0, si1)
    semr = (sr0, sr1)
    semw = (sw0, sw1)

    pltpu.sync_copy(hs_hbm.at[c, pl.ds(s * RPT, RPT)], hsm.at[pl.ds(s * RPT, RPT)])

    def zrow(r, _):
        for cc in range(DH // 16):
            zbuf[r, pl.ds(cc * 16, 16)] = jnp.zeros((16,), jnp.float32)
        return 0

    lax.fori_loop(0, RZ, zrow, 0)
    for k in range(RPT // RZ):
        pltpu.sync_copy(zbuf, acc.at[pl.ds(s * RPT + k * RZ, RZ)])
    plsc.subcore_barrier()

    rbase = s * (EPT // 128)

    def fetch(j, b):
        roff = rbase + j * Q
        pltpu.async_copy(src_hbm.at[pl.ds(roff, Q)], sidx.at[b], semi[b])
        pltpu.async_copy(dst_hbm.at[pl.ds(roff, Q)], didx.at[b], semi[b])
        pltpu.async_copy(ew_hbm.at[pl.ds(roff, Q)], ewb.at[b], semi[b])

    def drain_idx(b):
        pltpu.make_async_copy(src_hbm.at[pl.ds(0, Q)], sidx.at[b], semi[b]).wait()
        pltpu.make_async_copy(dst_hbm.at[pl.ds(0, Q)], didx.at[b], semi[b]).wait()
        pltpu.make_async_copy(ew_hbm.at[pl.ds(0, Q)], ewb.at[b], semi[b]).wait()

    def start_gather(b):
        for q in range(Q):
            pltpu.async_copy(hsm.at[sidx.at[b, q]],
                             rows.at[b, pl.ds(q * 128, 128)], semr[b])

    def drain_rows(b):
        for q in range(Q):
            pltpu.make_async_copy(hsm.at[sidx.at[b, q]],
                                  rows.at[b, pl.ds(q * 128, 128)],
                                  semr[b]).wait()

    def start_scatter(b):
        for q in range(Q):
            for i in range(8):
                didx_s[b, q, pl.ds(i * 16, 16)] = didx[b, q, pl.ds(i * 16, 16)]
        for q in range(Q):
            pltpu.async_copy(rows.at[b, pl.ds(q * 128, 128)],
                             acc.at[didx_s.at[b, q]], semw[b], add=True)

    def drain_scatter(b):
        for q in range(Q):
            pltpu.make_async_copy(rows.at[b, pl.ds(q * 128, 128)],
                                  acc.at[didx_s.at[b, q]], semw[b]).wait()

    fetch(0, 0)
    drain_idx(0)
    start_gather(0)
    fetch(1, 1)

    def body(jj, _):
        for b in (0, 1):
            j = jj * 2 + b
            drain_rows(b)

            @pl.when(jnp.logical_and(j >= 1, j + 1 < NCHUNK))
            def _():
                drain_scatter(1 - b)

            @pl.when(j + 1 < NCHUNK)
            def _():
                drain_idx(1 - b)
                start_gather(1 - b)

            def scale(g, _):
                q = g // 8
                g8 = g % 8
                wv = ewb[b, q, pl.ds(g8 * 16, 16)]
                for l in range(16):
                    w = wv[l]
                    e = g * 16 + l
                    for cc in range(DH // 16):
                        rows[b, e, pl.ds(cc * 16, 16)] = (
                            rows[b, e, pl.ds(cc * 16, 16)] * w)
                return 0

            lax.fori_loop(0, CH // 16, scale, 0)
            start_scatter(b)

            @pl.when(j + 2 < NCHUNK)
            def _():
                fetch(j + 2, b)
        return 0

    lax.fori_loop(0, NCHUNK // 2, body, 0)
    drain_scatter(0)
    drain_scatter(1)
    plsc.subcore_barrier()

    for k in range(RPT // RCH):
        r0 = s * RPT + k * RCH
        pltpu.sync_copy(acc.at[pl.ds(r0, RCH)], out_hbm.at[c, pl.ds(r0, RCH)])


_RB = 1000


def _m1_body(x_ref, w_ref, degp_ref, hs_ref, dinv_ref):
    deg = jnp.sum(degp_ref[...], axis=1) + 1.0
    dinv = lax.rsqrt(deg)
    h = lax.dot_general(x_ref[...], w_ref[...],
                        (((1,), (1,)), ((), ())),
                        preferred_element_type=jnp.float32)
    hs = h * dinv[:, None]
    hs_ref[0] = hs[:, :DH]
    hs_ref[1] = hs[:, DH:]
    dinv_ref[...] = dinv[:, None]


def _tc_stage1(x, w1, degp):
    return pl.pallas_call(
        _m1_body,
        grid=(N // _RB,),
        in_specs=[
            pl.BlockSpec((_RB, D), lambda i: (i, 0)),
            pl.BlockSpec((D, D), lambda i: (0, 0)),
            pl.BlockSpec((_RB, NC), lambda i: (i, 0)),
        ],
        out_specs=[
            pl.BlockSpec((NC, _RB, DH), lambda i: (0, i, 0)),
            pl.BlockSpec((_RB, 1), lambda i: (i, 0)),
        ],
        out_shape=[
            jax.ShapeDtypeStruct((NC, NPAD, DH), jnp.float32),
            jax.ShapeDtypeStruct((N, 1), jnp.float32),
        ],
    )(x, w1, degp)


def _m2_body(accp_ref, hsp_ref, dinv_ref, b_ref, w_ref, out_ref):
    dinv = dinv_ref[...]
    za = jnp.concatenate([accp_ref[0], accp_ref[1]], axis=1)
    zh = jnp.concatenate([hsp_ref[0], hsp_ref[1]], axis=1)
    z = dinv * (za + zh) + b_ref[...]
    r = jnp.maximum(z, 0.0)
    h2 = lax.dot_general(r, w_ref[...],
                         (((1,), (1,)), ((), ())),
                         preferred_element_type=jnp.float32)
    hs2 = h2 * dinv
    out_ref[0] = hs2[:, :DH]
    out_ref[1] = hs2[:, DH:]


def _tc_stage2(accp, hsp, dinv, b1, w2):
    return pl.pallas_call(
        _m2_body,
        grid=(N // _RB,),
        in_specs=[
            pl.BlockSpec((NC, _RB, DH), lambda i: (0, i, 0)),
            pl.BlockSpec((NC, _RB, DH), lambda i: (0, i, 0)),
            pl.BlockSpec((_RB, 1), lambda i: (i, 0)),
            pl.BlockSpec((1, D), lambda i: (0, 0)),
            pl.BlockSpec((D, D), lambda i: (0, 0)),
        ],
        out_specs=pl.BlockSpec((NC, _RB, DH), lambda i: (0, i, 0)),
        out_shape=jax.ShapeDtypeStruct((NC, NPAD, DH), jnp.float32),
    )(accp, hsp, dinv, b1, w2)


def _m3_body(accp_ref, hsp_ref, dinv_ref, b_ref, out_ref):
    dinv = dinv_ref[...]
    za = jnp.concatenate([accp_ref[0], accp_ref[1]], axis=1)
    zh = jnp.concatenate([hsp_ref[0], hsp_ref[1]], axis=1)
    out_ref[...] = dinv * (za + zh) + b_ref[...]


def _tc_stage3(accp, hsp, dinv, b2):
    return pl.pallas_call(
        _m3_body,
        grid=(N // _RB,),
        in_specs=[
            pl.BlockSpec((NC, _RB, DH), lambda i: (0, i, 0)),
            pl.BlockSpec((NC, _RB, DH), lambda i: (0, i, 0)),
            pl.BlockSpec((_RB, 1), lambda i: (i, 0)),
            pl.BlockSpec((1, D), lambda i: (0, 0)),
        ],
        out_specs=pl.BlockSpec((_RB, D), lambda i: (i, 0)),
        out_shape=jax.ShapeDtypeStruct((N, D), jnp.float32),
    )(accp, hsp, dinv, b2)


def kernel(x, edge_index, edge_weight, W1, b1, W2, b2):
    src = edge_index[0]
    dst = edge_index[1]
    pad = EP - E
    srcp = jnp.concatenate([src, jnp.zeros((pad,), src.dtype)])
    dstp = jnp.concatenate([dst, jnp.zeros((pad,), dst.dtype)])
    ewp = jnp.concatenate([edge_weight, jnp.zeros((pad,), edge_weight.dtype)])
    src2 = srcp.reshape(EP // 128, 128)
    dst2 = dstp.reshape(EP // 128, 128)
    ew2 = ewp.reshape(EP // 128, 128)
    b1r = b1.reshape(1, D)
    b2r = b2.reshape(1, D)

    degp = _deg_kernel(dstp, ewp)
    degt = degp.reshape(NC, NPAD).T
    hs1, dinv = _tc_stage1(x, W1, degt)
    acc1 = _msg_kernel(hs1, src2, dst2, ew2)
    hs2 = _tc_stage2(acc1, hs1, dinv, b1r, W2)
    acc2 = _msg_kernel(hs2, src2, dst2, ew2)
    out = _tc_stage3(acc2, hs2, dinv, b2r)
    return out


# speedup vs baseline: 1.5964x; 1.0178x over previous
import functools
import jax
import jax.numpy as jnp
from jax import lax
from jax.experimental import pallas as pl
from jax.experimental.pallas import tpu as pltpu
from jax.experimental.pallas import tpu_sc as plsc

N = 10000
E = 320000
D = 128
DH = D // 2

NC = 2
NS = 16
NPAD = 10240
EPT = 20480
EP = NS * EPT
CH = 256
Q = CH // 128
NCHUNK = EPT // CH
RPT = NPAD // NS
RCH = 128
RZ = 64

_SC_MESH = plsc.VectorSubcoreMesh(core_axis_name="c", subcore_axis_name="s")


@functools.partial(
    pl.kernel,
    out_type=jax.ShapeDtypeStruct((NC, 1, NPAD), jnp.float32),
    mesh=_SC_MESH,
    scratch_types=[
        pltpu.VMEM_SHARED((NPAD,), jnp.float32),
        pltpu.VMEM((2048,), jnp.float32),
        pltpu.VMEM((CH,), jnp.int32),
        pltpu.VMEM((CH,), jnp.float32),
        pltpu.SemaphoreType.DMA,
    ],
)
def _deg_kernel(dst_hbm, ew_hbm, out_hbm, acc, zbuf, didx, wbuf, sem):
    c = lax.axis_index("c")
    s = lax.axis_index("s")
    wid = c * NS + s

    @pl.when(s == 0)
    def _():
        def z(i, _):
            zbuf[pl.ds(i * 16, 16)] = jnp.zeros((16,), jnp.float32)
            return 0

        lax.fori_loop(0, 2048 // 16, z, 0)
        for k in range(NPAD // 2048):
            pltpu.sync_copy(zbuf, acc.at[pl.ds(k * 2048, 2048)])

    plsc.subcore_barrier()

    ebase = wid * (EP // (NC * NS))

    def chunk(j, _):
        off = ebase + j * CH
        cp1 = pltpu.async_copy(dst_hbm.at[pl.ds(off, CH)], didx, sem)
        cp2 = pltpu.async_copy(ew_hbm.at[pl.ds(off, CH)], wbuf, sem)
        cp1.wait()
        cp2.wait()
        pltpu.sync_copy(wbuf, acc.at[didx], add=True)
        return 0

    lax.fori_loop(0, EP // (NC * NS) // CH, chunk, 0)
    plsc.subcore_barrier()

    @pl.when(s == 0)
    def _():
        pltpu.sync_copy(acc, out_hbm.at[c, 0])


@functools.partial(
    pl.kernel,
    out_type=jax.ShapeDtypeStruct((NC, NPAD, DH), jnp.float32),
    mesh=_SC_MESH,
    compiler_params=pltpu.CompilerParams(use_tc_tiling_on_sc=False),
    scratch_types=[
        pltpu.VMEM_SHARED((NPAD, DH), jnp.float32),
        pltpu.VMEM_SHARED((NPAD, DH), jnp.float32),
        pltpu.VMEM((RZ, DH), jnp.float32),
        pltpu.VMEM((2, 2, CH), jnp.int32),
        pltpu.VMEM((2, CH), jnp.float32),
        pltpu.VMEM((2, CH), jnp.int32),
        pltpu.VMEM((2, CH, DH), jnp.float32),
        pltpu.SemaphoreType.DMA,
        pltpu.SemaphoreType.DMA,
        pltpu.SemaphoreType.DMA,
        pltpu.SemaphoreType.DMA,
        pltpu.SemaphoreType.DMA,
        pltpu.SemaphoreType.DMA,
    ],
)
def _msg_kernel(hs_hbm, ed_hbm, ew_hbm, out_hbm,
                acc, hsm, zbuf, ebuf, ewb, didx_s, rows,
                si0, si1, sr0, sr1, sw0, sw1):
    c = lax.axis_index("c")
    s = lax.axis_index("s")
    semi = (si0, si1)
    semr = (sr0, sr1)
    semw = (sw0, sw1)

    pltpu.sync_copy(hs_hbm.at[c, pl.ds(s * RPT, RPT)], hsm.at[pl.ds(s * RPT, RPT)])

    def zrow(r, _):
        for cc in range(DH // 16):
            zbuf[r, pl.ds(cc * 16, 16)] = jnp.zeros((16,), jnp.float32)
        return 0

    lax.fori_loop(0, RZ, zrow, 0)
    for k in range(RPT // RZ):
        pltpu.sync_copy(zbuf, acc.at[pl.ds(s * RPT + k * RZ, RZ)])
    plsc.subcore_barrier()

    cbase = s * NCHUNK

    def fetch(j, b):
        pltpu.async_copy(ed_hbm.at[cbase + j], ebuf.at[b], semi[b])
        pltpu.async_copy(ew_hbm.at[cbase + j], ewb.at[b], semi[b])

    def drain_idx(b):
        pltpu.make_async_copy(ed_hbm.at[0], ebuf.at[b], semi[b]).wait()
        pltpu.make_async_copy(ew_hbm.at[0], ewb.at[b], semi[b]).wait()

    def start_gather(b):
        pltpu.async_copy(hsm.at[ebuf.at[b, 0]], rows.at[b], semr[b])

    def drain_rows(b):
        pltpu.make_async_copy(hsm.at[ebuf.at[b, 0]], rows.at[b], semr[b]).wait()

    def start_scatter(b):
        for i in range(CH // 16):
            didx_s[b, pl.ds(i * 16, 16)] = ebuf[b, 1, pl.ds(i * 16, 16)]
        pltpu.async_copy(rows.at[b], acc.at[didx_s.at[b]], semw[b], add=True)

    def drain_scatter(b):
        pltpu.make_async_copy(rows.at[b], acc.at[didx_s.at[b]], semw[b]).wait()

    fetch(0, 0)
    drain_idx(0)
    start_gather(0)
    fetch(1, 1)

    def body(jj, _):
        for b in (0, 1):
            j = jj * 2 + b
            drain_rows(b)

            @pl.when(jnp.logical_and(j >= 1, j + 1 < NCHUNK))
            def _():
                drain_scatter(1 - b)

            @pl.when(j + 1 < NCHUNK)
            def _():
                drain_idx(1 - b)
                start_gather(1 - b)

            def scale(g, _):
                wv = ewb[b, pl.ds(g * 16, 16)]
                for l in range(16):
                    w = wv[l]
                    e = g * 16 + l
                    for cc in range(DH // 16):
                        rows[b, e, pl.ds(cc * 16, 16)] = (
                            rows[b, e, pl.ds(cc * 16, 16)] * w)
                return 0

            lax.fori_loop(0, CH // 16, scale, 0)
            start_scatter(b)

            @pl.when(j + 2 < NCHUNK)
            def _():
                fetch(j + 2, b)
        return 0

    lax.fori_loop(0, NCHUNK // 2, body, 0)
    drain_scatter(0)
    drain_scatter(1)
    plsc.subcore_barrier()

    for k in range(RPT // RCH):
        r0 = s * RPT + k * RCH
        pltpu.sync_copy(acc.at[pl.ds(r0, RCH)], out_hbm.at[c, pl.ds(r0, RCH)])


_RB = 1000


def _m1_body(x_ref, w_ref, degp_ref, hs_ref, dinv_ref):
    deg = jnp.sum(degp_ref[...], axis=1) + 1.0
    dinv = lax.rsqrt(deg)
    h = lax.dot_general(x_ref[...], w_ref[...],
                        (((1,), (1,)), ((), ())),
                        preferred_element_type=jnp.float32)
    hs = h * dinv[:, None]
    hs_ref[0] = hs[:, :DH]
    hs_ref[1] = hs[:, DH:]
    dinv_ref[...] = dinv[:, None]


def _tc_stage1(x, w1, degp):
    return pl.pallas_call(
        _m1_body,
        grid=(N // _RB,),
        in_specs=[
            pl.BlockSpec((_RB, D), lambda i: (i, 0)),
            pl.BlockSpec((D, D), lambda i: (0, 0)),
            pl.BlockSpec((_RB, NC), lambda i: (i, 0)),
        ],
        out_specs=[
            pl.BlockSpec((NC, _RB, DH), lambda i: (0, i, 0)),
            pl.BlockSpec((_RB, 1), lambda i: (i, 0)),
        ],
        out_shape=[
            jax.ShapeDtypeStruct((NC, NPAD, DH), jnp.float32),
            jax.ShapeDtypeStruct((N, 1), jnp.float32),
        ],
    )(x, w1, degp)


def _m2_body(accp_ref, hsp_ref, dinv_ref, b_ref, w_ref, out_ref):
    dinv = dinv_ref[...]
    za = jnp.concatenate([accp_ref[0], accp_ref[1]], axis=1)
    zh = jnp.concatenate([hsp_ref[0], hsp_ref[1]], axis=1)
    z = dinv * (za + zh) + b_ref[...]
    r = jnp.maximum(z, 0.0)
    h2 = lax.dot_general(r, w_ref[...],
                         (((1,), (1,)), ((), ())),
                         preferred_element_type=jnp.float32)
    hs2 = h2 * dinv
    out_ref[0] = hs2[:, :DH]
    out_ref[1] = hs2[:, DH:]


def _tc_stage2(accp, hsp, dinv, b1, w2):
    return pl.pallas_call(
        _m2_body,
        grid=(N // _RB,),
        in_specs=[
            pl.BlockSpec((NC, _RB, DH), lambda i: (0, i, 0)),
            pl.BlockSpec((NC, _RB, DH), lambda i: (0, i, 0)),
            pl.BlockSpec((_RB, 1), lambda i: (i, 0)),
            pl.BlockSpec((1, D), lambda i: (0, 0)),
            pl.BlockSpec((D, D), lambda i: (0, 0)),
        ],
        out_specs=pl.BlockSpec((NC, _RB, DH), lambda i: (0, i, 0)),
        out_shape=jax.ShapeDtypeStruct((NC, NPAD, DH), jnp.float32),
    )(accp, hsp, dinv, b1, w2)


def _m3_body(accp_ref, hsp_ref, dinv_ref, b_ref, out_ref):
    dinv = dinv_ref[...]
    za = jnp.concatenate([accp_ref[0], accp_ref[1]], axis=1)
    zh = jnp.concatenate([hsp_ref[0], hsp_ref[1]], axis=1)
    out_ref[...] = dinv * (za + zh) + b_ref[...]


def _tc_stage3(accp, hsp, dinv, b2):
    return pl.pallas_call(
        _m3_body,
        grid=(N // _RB,),
        in_specs=[
            pl.BlockSpec((NC, _RB, DH), lambda i: (0, i, 0)),
            pl.BlockSpec((NC, _RB, DH), lambda i: (0, i, 0)),
            pl.BlockSpec((_RB, 1), lambda i: (i, 0)),
            pl.BlockSpec((1, D), lambda i: (0, 0)),
        ],
        out_specs=pl.BlockSpec((_RB, D), lambda i: (i, 0)),
        out_shape=jax.ShapeDtypeStruct((N, D), jnp.float32),
    )(accp, hsp, dinv, b2)


def kernel(x, edge_index, edge_weight, W1, b1, W2, b2):
    src = edge_index[0]
    dst = edge_index[1]
    pad = EP - E
    srcp = jnp.concatenate([src, jnp.zeros((pad,), src.dtype)])
    dstp = jnp.concatenate([dst, jnp.zeros((pad,), dst.dtype)])
    ewp = jnp.concatenate([edge_weight, jnp.zeros((pad,), edge_weight.dtype)])
    edata = jnp.stack([srcp.reshape(EP // CH, CH),
                       dstp.reshape(EP // CH, CH)], axis=1)
    ewd = ewp.reshape(EP // CH, CH)
    b1r = b1.reshape(1, D)
    b2r = b2.reshape(1, D)

    degp = _deg_kernel(dstp, ewp)
    degt = degp.reshape(NC, NPAD).T
    hs1, dinv = _tc_stage1(x, W1, degt)
    acc1 = _msg_kernel(hs1, edata, ewd)
    hs2 = _tc_stage2(acc1, hs1, dinv, b1r, W2)
    acc2 = _msg_kernel(hs2, edata, ewd)
    out = _tc_stage3(acc2, hs2, dinv, b2r)
    return out


# speedup vs baseline: 1.6181x; 1.0136x over previous
import functools
import jax
import jax.numpy as jnp
from jax import lax
from jax.experimental import pallas as pl
from jax.experimental.pallas import tpu as pltpu
from jax.experimental.pallas import tpu_sc as plsc

N = 10000
E = 320000
D = 128
DH = D // 2

NC = 2
NS = 16
NPAD = 10240
EPT = 20480
EP = NS * EPT
CH = 256
Q = CH // 128
NCHUNK = EPT // CH
RPT = NPAD // NS
RCH = 128
RZ = 64

_SC_MESH = plsc.VectorSubcoreMesh(core_axis_name="c", subcore_axis_name="s")


@functools.partial(
    pl.kernel,
    out_type=jax.ShapeDtypeStruct((NC, 1, NPAD), jnp.float32),
    mesh=_SC_MESH,
    compiler_params=pltpu.CompilerParams(use_tc_tiling_on_sc=False),
    scratch_types=[
        pltpu.VMEM_SHARED((NPAD,), jnp.float32),
        pltpu.VMEM((2048,), jnp.float32),
        pltpu.VMEM((2, CH), jnp.int32),
        pltpu.VMEM((2, CH), jnp.float32),
        pltpu.SemaphoreType.DMA,
    ],
)
def _deg_kernel(dst_hbm, ew_hbm, out_hbm, acc, zbuf, didx, wbuf, sem):
    c = lax.axis_index("c")
    s = lax.axis_index("s")
    wid = c * NS + s

    @pl.when(s == 0)
    def _():
        def z(i, _):
            zbuf[pl.ds(i * 16, 16)] = jnp.zeros((16,), jnp.float32)
            return 0

        lax.fori_loop(0, 2048 // 16, z, 0)
        for k in range(NPAD // 2048):
            pltpu.sync_copy(zbuf, acc.at[pl.ds(k * 2048, 2048)])

    plsc.subcore_barrier()

    ebase = wid * (EP // (NC * NS))
    ndchunk = EP // (NC * NS) // CH

    def dfetch(j, b):
        off = ebase + j * CH
        pltpu.async_copy(dst_hbm.at[pl.ds(off, CH)], didx.at[b], sem)
        pltpu.async_copy(ew_hbm.at[pl.ds(off, CH)], wbuf.at[b], sem)

    def ddrain(b):
        pltpu.make_async_copy(dst_hbm.at[pl.ds(0, CH)], didx.at[b], sem).wait()
        pltpu.make_async_copy(ew_hbm.at[pl.ds(0, CH)], wbuf.at[b], sem).wait()

    dfetch(0, 0)
    dfetch(1, 1)

    def chunk(jj, _):
        for b in (0, 1):
            j = jj * 2 + b
            ddrain(b)
            pltpu.sync_copy(wbuf.at[b], acc.at[didx.at[b]], add=True)

            @pl.when(j + 2 < ndchunk)
            def _():
                dfetch(j + 2, b)
        return 0

    lax.fori_loop(0, ndchunk // 2, chunk, 0)
    plsc.subcore_barrier()

    @pl.when(s == 0)
    def _():
        pltpu.sync_copy(acc, out_hbm.at[c, 0])


@functools.partial(
    pl.kernel,
    out_type=jax.ShapeDtypeStruct((NC, NPAD, DH), jnp.float32),
    mesh=_SC_MESH,
    compiler_params=pltpu.CompilerParams(use_tc_tiling_on_sc=False),
    scratch_types=[
        pltpu.VMEM_SHARED((NPAD, DH), jnp.float32),
        pltpu.VMEM_SHARED((NPAD, DH), jnp.float32),
        pltpu.VMEM((RZ, DH), jnp.float32),
        pltpu.VMEM((2, 2, CH), jnp.int32),
        pltpu.VMEM((2, CH), jnp.float32),
        pltpu.VMEM((2, CH), jnp.int32),
        pltpu.VMEM((2, CH, DH), jnp.float32),
        pltpu.SemaphoreType.DMA,
        pltpu.SemaphoreType.DMA,
        pltpu.SemaphoreType.DMA,
        pltpu.SemaphoreType.DMA,
        pltpu.SemaphoreType.DMA,
        pltpu.SemaphoreType.DMA,
    ],
)
def _msg_kernel(hs_hbm, ed_hbm, ew_hbm, out_hbm,
                acc, hsm, zbuf, ebuf, ewb, didx_s, rows,
                si0, si1, sr0, sr1, sw0, sw1):
    c = lax.axis_index("c")
    s = lax.axis_index("s")
    semi = (si0, si1)
    semr = (sr0, sr1)
    semw = (sw0, sw1)

    pltpu.sync_copy(hs_hbm.at[c, pl.ds(s * RPT, RPT)], hsm.at[pl.ds(s * RPT, RPT)])

    def zrow(r, _):
        for cc in range(DH // 16):
            zbuf[r, pl.ds(cc * 16, 16)] = jnp.zeros((16,), jnp.float32)
        return 0

    lax.fori_loop(0, RZ, zrow, 0)
    for k in range(RPT // RZ):
        pltpu.sync_copy(zbuf, acc.at[pl.ds(s * RPT + k * RZ, RZ)])
    plsc.subcore_barrier()

    cbase = s * NCHUNK

    def fetch(j, b):
        pltpu.async_copy(ed_hbm.at[cbase + j], ebuf.at[b], semi[b])
        pltpu.async_copy(ew_hbm.at[cbase + j], ewb.at[b], semi[b])

    def drain_idx(b):
        pltpu.make_async_copy(ed_hbm.at[0], ebuf.at[b], semi[b]).wait()
        pltpu.make_async_copy(ew_hbm.at[0], ewb.at[b], semi[b]).wait()

    def start_gather(b):
        pltpu.async_copy(hsm.at[ebuf.at[b, 0]], rows.at[b], semr[b])

    def drain_rows(b):
        pltpu.make_async_copy(hsm.at[ebuf.at[b, 0]], rows.at[b], semr[b]).wait()

    def start_scatter(b):
        for i in range(CH // 16):
            didx_s[b, pl.ds(i * 16, 16)] = ebuf[b, 1, pl.ds(i * 16, 16)]
        pltpu.async_copy(rows.at[b], acc.at[didx_s.at[b]], semw[b], add=True)

    def drain_scatter(b):
        pltpu.make_async_copy(rows.at[b], acc.at[didx_s.at[b]], semw[b]).wait()

    fetch(0, 0)
    drain_idx(0)
    start_gather(0)
    fetch(1, 1)

    def body(jj, _):
        for b in (0, 1):
            j = jj * 2 + b
            drain_rows(b)

            @pl.when(jnp.logical_and(j >= 1, j + 1 < NCHUNK))
            def _():
                drain_scatter(1 - b)

            @pl.when(j + 1 < NCHUNK)
            def _():
                drain_idx(1 - b)
                start_gather(1 - b)

            def scale(g, _):
                wv = ewb[b, pl.ds(g * 16, 16)]
                for l in range(16):
                    w = wv[l]
                    e = g * 16 + l
                    for cc in range(DH // 16):
                        rows[b, e, pl.ds(cc * 16, 16)] = (
                            rows[b, e, pl.ds(cc * 16, 16)] * w)
                return 0

            lax.fori_loop(0, CH // 16, scale, 0)
            start_scatter(b)

            @pl.when(j + 2 < NCHUNK)
            def _():
                fetch(j + 2, b)
        return 0

    lax.fori_loop(0, NCHUNK // 2, body, 0)
    drain_scatter(0)
    drain_scatter(1)
    plsc.subcore_barrier()

    for k in range(RPT // RCH):
        r0 = s * RPT + k * RCH
        pltpu.sync_copy(acc.at[pl.ds(r0, RCH)], out_hbm.at[c, pl.ds(r0, RCH)])


_RB = 1000


def _mm1_body(x_ref, w_ref, h_ref):
    h_ref[...] = lax.dot_general(x_ref[...], w_ref[...],
                                 (((1,), (1,)), ((), ())),
                                 preferred_element_type=jnp.float32)


def _tc_mm1(x, w1):
    return pl.pallas_call(
        _mm1_body,
        grid=(N // _RB,),
        in_specs=[
            pl.BlockSpec((_RB, D), lambda i: (i, 0)),
            pl.BlockSpec((D, D), lambda i: (0, 0)),
        ],
        out_specs=pl.BlockSpec((_RB, D), lambda i: (i, 0)),
        out_shape=jax.ShapeDtypeStruct((N, D), jnp.float32),
    )(x, w1)


def _m1_body(h_ref, degp_ref, hs_ref, dinv_ref):
    deg = jnp.sum(degp_ref[...], axis=1) + 1.0
    dinv = lax.rsqrt(deg)
    hs = h_ref[...] * dinv[:, None]
    hs_ref[0] = hs[:, :DH]
    hs_ref[1] = hs[:, DH:]
    dinv_ref[...] = dinv[:, None]


def _tc_stage1(h, degp):
    return pl.pallas_call(
        _m1_body,
        grid=(N // _RB,),
        in_specs=[
            pl.BlockSpec((_RB, D), lambda i: (i, 0)),
            pl.BlockSpec((_RB, NC), lambda i: (i, 0)),
        ],
        out_specs=[
            pl.BlockSpec((NC, _RB, DH), lambda i: (0, i, 0)),
            pl.BlockSpec((_RB, 1), lambda i: (i, 0)),
        ],
        out_shape=[
            jax.ShapeDtypeStruct((NC, NPAD, DH), jnp.float32),
            jax.ShapeDtypeStruct((N, 1), jnp.float32),
        ],
    )(h, degp)


def _m2_body(accp_ref, hsp_ref, dinv_ref, b_ref, w_ref, out_ref):
    dinv = dinv_ref[...]
    za = jnp.concatenate([accp_ref[0], accp_ref[1]], axis=1)
    zh = jnp.concatenate([hsp_ref[0], hsp_ref[1]], axis=1)
    z = dinv * (za + zh) + b_ref[...]
    r = jnp.maximum(z, 0.0)
    h2 = lax.dot_general(r, w_ref[...],
                         (((1,), (1,)), ((), ())),
                         preferred_element_type=jnp.float32)
    hs2 = h2 * dinv
    out_ref[0] = hs2[:, :DH]
    out_ref[1] = hs2[:, DH:]


def _tc_stage2(accp, hsp, dinv, b1, w2):
    return pl.pallas_call(
        _m2_body,
        grid=(N // _RB,),
        in_specs=[
            pl.BlockSpec((NC, _RB, DH), lambda i: (0, i, 0)),
            pl.BlockSpec((NC, _RB, DH), lambda i: (0, i, 0)),
            pl.BlockSpec((_RB, 1), lambda i: (i, 0)),
            pl.BlockSpec((1, D), lambda i: (0, 0)),
            pl.BlockSpec((D, D), lambda i: (0, 0)),
        ],
        out_specs=pl.BlockSpec((NC, _RB, DH), lambda i: (0, i, 0)),
        out_shape=jax.ShapeDtypeStruct((NC, NPAD, DH), jnp.float32),
    )(accp, hsp, dinv, b1, w2)


def _m3_body(accp_ref, hsp_ref, dinv_ref, b_ref, out_ref):
    dinv = dinv_ref[...]
    za = jnp.concatenate([accp_ref[0], accp_ref[1]], axis=1)
    zh = jnp.concatenate([hsp_ref[0], hsp_ref[1]], axis=1)
    out_ref[...] = dinv * (za + zh) + b_ref[...]


def _tc_stage3(accp, hsp, dinv, b2):
    return pl.pallas_call(
        _m3_body,
        grid=(N // _RB,),
        in_specs=[
            pl.BlockSpec((NC, _RB, DH), lambda i: (0, i, 0)),
            pl.BlockSpec((NC, _RB, DH), lambda i: (0, i, 0)),
            pl.BlockSpec((_RB, 1), lambda i: (i, 0)),
            pl.BlockSpec((1, D), lambda i: (0, 0)),
        ],
        out_specs=pl.BlockSpec((_RB, D), lambda i: (i, 0)),
        out_shape=jax.ShapeDtypeStruct((N, D), jnp.float32),
    )(accp, hsp, dinv, b2)


def kernel(x, edge_index, edge_weight, W1, b1, W2, b2):
    src = edge_index[0]
    dst = edge_index[1]
    pad = EP - E
    srcp = jnp.concatenate([src, jnp.zeros((pad,), src.dtype)])
    dstp = jnp.concatenate([dst, jnp.zeros((pad,), dst.dtype)])
    ewp = jnp.concatenate([edge_weight, jnp.zeros((pad,), edge_weight.dtype)])
    edata = jnp.stack([srcp.reshape(EP // CH, CH),
                       dstp.reshape(EP // CH, CH)], axis=1)
    ewd = ewp.reshape(EP // CH, CH)
    b1r = b1.reshape(1, D)
    b2r = b2.reshape(1, D)

    degp = _deg_kernel(dstp, ewp)
    h1 = _tc_mm1(x, W1)
    degt = degp.reshape(NC, NPAD).T
    hs1, dinv = _tc_stage1(h1, degt)
    acc1 = _msg_kernel(hs1, edata, ewd)
    hs2 = _tc_stage2(acc1, hs1, dinv, b1r, W2)
    acc2 = _msg_kernel(hs2, edata, ewd)
    out = _tc_stage3(acc2, hs2, dinv, b2r)
    return out


# speedup vs baseline: 1.6357x; 1.0109x over previous
import functools
import jax
import jax.numpy as jnp
from jax import lax
from jax.experimental import pallas as pl
from jax.experimental.pallas import tpu as pltpu
from jax.experimental.pallas import tpu_sc as plsc

N = 10000
E = 320000
D = 128
DH = D // 2

NC = 2
NS = 16
NPAD = 10240
EPT = 20480
EP = NS * EPT
CH = 256
Q = CH // 128
NCHUNK = EPT // CH
RPT = NPAD // NS
RCH = 128
RZ = 64

_SC_MESH = plsc.VectorSubcoreMesh(core_axis_name="c", subcore_axis_name="s",
                                  num_cores=NC, num_subcores=NS)


@functools.partial(
    pl.kernel,
    out_type=jax.ShapeDtypeStruct((NC, 1, NPAD), jnp.float32),
    mesh=_SC_MESH,
    compiler_params=pltpu.CompilerParams(use_tc_tiling_on_sc=False),
    scratch_types=[
        pltpu.VMEM_SHARED((NPAD,), jnp.float32),
        pltpu.VMEM((2048,), jnp.float32),
        pltpu.VMEM((2, CH), jnp.int32),
        pltpu.VMEM((2, CH), jnp.float32),
        pltpu.SemaphoreType.DMA,
    ],
)
def _deg_kernel(dst_hbm, ew_hbm, out_hbm, acc, zbuf, didx, wbuf, sem):
    c = lax.axis_index("c")
    s = lax.axis_index("s")
    wid = c * NS + s

    @pl.when(s == 0)
    def _():
        def z(i, _):
            zbuf[pl.ds(i * 16, 16)] = jnp.zeros((16,), jnp.float32)
            return 0

        lax.fori_loop(0, 2048 // 16, z, 0)
        for k in range(NPAD // 2048):
            pltpu.sync_copy(zbuf, acc.at[pl.ds(k * 2048, 2048)])

    plsc.subcore_barrier()

    ebase = wid * (EP // (NC * NS))
    ndchunk = EP // (NC * NS) // CH

    def dfetch(j, b):
        off = ebase + j * CH
        pltpu.async_copy(dst_hbm.at[pl.ds(off, CH)], didx.at[b], sem)
        pltpu.async_copy(ew_hbm.at[pl.ds(off, CH)], wbuf.at[b], sem)

    def ddrain(b):
        pltpu.make_async_copy(dst_hbm.at[pl.ds(0, CH)], didx.at[b], sem).wait()
        pltpu.make_async_copy(ew_hbm.at[pl.ds(0, CH)], wbuf.at[b], sem).wait()

    dfetch(0, 0)
    dfetch(1, 1)

    def chunk(jj, _):
        for b in (0, 1):
            j = jj * 2 + b
            ddrain(b)
            pltpu.sync_copy(wbuf.at[b], acc.at[didx.at[b]], add=True)

            @pl.when(j + 2 < ndchunk)
            def _():
                dfetch(j + 2, b)
        return 0

    lax.fori_loop(0, ndchunk // 2, chunk, 0)
    plsc.subcore_barrier()

    @pl.when(s == 0)
    def _():
        pltpu.sync_copy(acc, out_hbm.at[c, 0])


@functools.partial(
    pl.kernel,
    out_type=jax.ShapeDtypeStruct((NC, NPAD, DH), jnp.float32),
    mesh=_SC_MESH,
    compiler_params=pltpu.CompilerParams(use_tc_tiling_on_sc=False),
    scratch_types=[
        pltpu.VMEM_SHARED((NPAD, DH), jnp.float32),
        pltpu.VMEM_SHARED((NPAD, DH), jnp.float32),
        pltpu.VMEM((RZ, DH), jnp.float32),
        pltpu.VMEM((2, 2, CH), jnp.int32),
        pltpu.VMEM((2, CH), jnp.float32),
        pltpu.VMEM((2, CH), jnp.int32),
        pltpu.VMEM((2, CH, DH), jnp.float32),
        pltpu.SemaphoreType.DMA,
        pltpu.SemaphoreType.DMA,
        pltpu.SemaphoreType.DMA,
        pltpu.SemaphoreType.DMA,
        pltpu.SemaphoreType.DMA,
        pltpu.SemaphoreType.DMA,
    ],
)
def _msg_kernel(hs_hbm, ed_hbm, ew_hbm, out_hbm,
                acc, hsm, zbuf, ebuf, ewb, didx_s, rows,
                si0, si1, sr0, sr1, sw0, sw1):
    c = lax.axis_index("c")
    s = lax.axis_index("s")
    semi = (si0, si1)
    semr = (sr0, sr1)
    semw = (sw0, sw1)

    pltpu.sync_copy(hs_hbm.at[c, pl.ds(s * RPT, RPT)], hsm.at[pl.ds(s * RPT, RPT)])

    def zrow(r, _):
        for cc in range(DH // 16):
            zbuf[r, pl.ds(cc * 16, 16)] = jnp.zeros((16,), jnp.float32)
        return 0

    lax.fori_loop(0, RZ, zrow, 0)
    for k in range(RPT // RZ):
        pltpu.sync_copy(zbuf, acc.at[pl.ds(s * RPT + k * RZ, RZ)])
    plsc.subcore_barrier()

    cbase = s * NCHUNK

    def fetch(j, b):
        pltpu.async_copy(ed_hbm.at[cbase + j], ebuf.at[b], semi[b])
        pltpu.async_copy(ew_hbm.at[cbase + j], ewb.at[b], semi[b])

    def drain_idx(b):
        pltpu.make_async_copy(ed_hbm.at[0], ebuf.at[b], semi[b]).wait()
        pltpu.make_async_copy(ew_hbm.at[0], ewb.at[b], semi[b]).wait()

    def start_gather(b):
        pltpu.async_copy(hsm.at[ebuf.at[b, 0]], rows.at[b], semr[b])

    def drain_rows(b):
        pltpu.make_async_copy(hsm.at[ebuf.at[b, 0]], rows.at[b], semr[b]).wait()

    def start_scatter(b):
        for i in range(CH // 16):
            didx_s[b, pl.ds(i * 16, 16)] = ebuf[b, 1, pl.ds(i * 16, 16)]
        pltpu.async_copy(rows.at[b], acc.at[didx_s.at[b]], semw[b], add=True)

    def drain_scatter(b):
        pltpu.make_async_copy(rows.at[b], acc.at[didx_s.at[b]], semw[b]).wait()

    fetch(0, 0)
    drain_idx(0)
    start_gather(0)
    fetch(1, 1)

    def body(jj, _):
        for b in (0, 1):
            j = jj * 2 + b
            drain_rows(b)

            @pl.when(jnp.logical_and(j >= 1, j + 1 < NCHUNK))
            def _():
                drain_scatter(1 - b)

            @pl.when(j + 1 < NCHUNK)
            def _():
                drain_idx(1 - b)
                start_gather(1 - b)

            def scale(g, _):
                wv = ewb[b, pl.ds(g * 16, 16)]
                for l in range(16):
                    w = wv[l]
                    e = g * 16 + l
                    for cc in range(DH // 16):
                        rows[b, e, pl.ds(cc * 16, 16)] = (
                            rows[b, e, pl.ds(cc * 16, 16)] * w)
                return 0

            lax.fori_loop(0, CH // 16, scale, 0)
            start_scatter(b)

            @pl.when(j + 2 < NCHUNK)
            def _():
                fetch(j + 2, b)
        return 0

    lax.fori_loop(0, NCHUNK // 2, body, 0)
    drain_scatter(0)
    drain_scatter(1)
    plsc.subcore_barrier()

    for k in range(RPT // RCH):
        r0 = s * RPT + k * RCH
        pltpu.sync_copy(acc.at[pl.ds(r0, RCH)], out_hbm.at[c, pl.ds(r0, RCH)])


_RB = 1000


def _mm1_body(x_ref, w_ref, h_ref):
    h_ref[...] = lax.dot_general(x_ref[...], w_ref[...],
                                 (((1,), (1,)), ((), ())),
                                 preferred_element_type=jnp.float32)


def _tc_mm1(x, w1):
    return pl.pallas_call(
        _mm1_body,
        grid=(N // _RB,),
        in_specs=[
            pl.BlockSpec((_RB, D), lambda i: (i, 0)),
            pl.BlockSpec((D, D), lambda i: (0, 0)),
        ],
        out_specs=pl.BlockSpec((_RB, D), lambda i: (i, 0)),
        out_shape=jax.ShapeDtypeStruct((N, D), jnp.float32),
    )(x, w1)


def _m1_body(h_ref, degp_ref, hs_ref, dinv_ref):
    deg = jnp.sum(degp_ref[...], axis=1) + 1.0
    dinv = lax.rsqrt(deg)
    hs = h_ref[...] * dinv[:, None]
    hs_ref[0] = hs[:, :DH]
    hs_ref[1] = hs[:, DH:]
    dinv_ref[...] = dinv[:, None]


def _tc_stage1(h, degp):
    return pl.pallas_call(
        _m1_body,
        grid=(N // _RB,),
        in_specs=[
            pl.BlockSpec((_RB, D), lambda i: (i, 0)),
            pl.BlockSpec((_RB, NC), lambda i: (i, 0)),
        ],
        out_specs=[
            pl.BlockSpec((NC, _RB, DH), lambda i: (0, i, 0)),
            pl.BlockSpec((_RB, 1), lambda i: (i, 0)),
        ],
        out_shape=[
            jax.ShapeDtypeStruct((NC, NPAD, DH), jnp.float32),
            jax.ShapeDtypeStruct((N, 1), jnp.float32),
        ],
    )(h, degp)


def _m2_body(accp_ref, hsp_ref, dinv_ref, b_ref, w_ref, out_ref):
    dinv = dinv_ref[...]
    za = jnp.concatenate([accp_ref[0], accp_ref[1]], axis=1)
    zh = jnp.concatenate([hsp_ref[0], hsp_ref[1]], axis=1)
    z = dinv * (za + zh) + b_ref[...]
    r = jnp.maximum(z, 0.0)
    h2 = lax.dot_general(r, w_ref[...],
                         (((1,), (1,)), ((), ())),
                         preferred_element_type=jnp.float32)
    hs2 = h2 * dinv
    out_ref[0] = hs2[:, :DH]
    out_ref[1] = hs2[:, DH:]


def _tc_stage2(accp, hsp, dinv, b1, w2):
    return pl.pallas_call(
        _m2_body,
        grid=(N // _RB,),
        in_specs=[
            pl.BlockSpec((NC, _RB, DH), lambda i: (0, i, 0)),
            pl.BlockSpec((NC, _RB, DH), lambda i: (0, i, 0)),
            pl.BlockSpec((_RB, 1), lambda i: (i, 0)),
            pl.BlockSpec((1, D), lambda i: (0, 0)),
            pl.BlockSpec((D, D), lambda i: (0, 0)),
        ],
        out_specs=pl.BlockSpec((NC, _RB, DH), lambda i: (0, i, 0)),
        out_shape=jax.ShapeDtypeStruct((NC, NPAD, DH), jnp.float32),
    )(accp, hsp, dinv, b1, w2)


def _m3_body(accp_ref, hsp_ref, dinv_ref, b_ref, out_ref):
    dinv = dinv_ref[...]
    za = jnp.concatenate([accp_ref[0], accp_ref[1]], axis=1)
    zh = jnp.concatenate([hsp_ref[0], hsp_ref[1]], axis=1)
    out_ref[...] = dinv * (za + zh) + b_ref[...]


def _tc_stage3(accp, hsp, dinv, b2):
    return pl.pallas_call(
        _m3_body,
        grid=(N // _RB,),
        in_specs=[
            pl.BlockSpec((NC, _RB, DH), lambda i: (0, i, 0)),
            pl.BlockSpec((NC, _RB, DH), lambda i: (0, i, 0)),
            pl.BlockSpec((_RB, 1), lambda i: (i, 0)),
            pl.BlockSpec((1, D), lambda i: (0, 0)),
        ],
        out_specs=pl.BlockSpec((_RB, D), lambda i: (i, 0)),
        out_shape=jax.ShapeDtypeStruct((N, D), jnp.float32),
    )(accp, hsp, dinv, b2)


def kernel(x, edge_index, edge_weight, W1, b1, W2, b2):
    src = edge_index[0]
    dst = edge_index[1]
    pad = EP - E
    srcp = jnp.concatenate([src, jnp.zeros((pad,), src.dtype)])
    dstp = jnp.concatenate([dst, jnp.zeros((pad,), dst.dtype)])
    ewp = jnp.concatenate([edge_weight, jnp.zeros((pad,), edge_weight.dtype)])
    edata = jnp.stack([srcp.reshape(EP // CH, CH),
                       dstp.reshape(EP // CH, CH)], axis=1)
    ewd = ewp.reshape(EP // CH, CH)
    b1r = b1.reshape(1, D)
    b2r = b2.reshape(1, D)

    degp = _deg_kernel(dstp, ewp)
    h1 = _tc_mm1(x, W1)
    degt = degp.reshape(NC, NPAD).T
    hs1, dinv = _tc_stage1(h1, degt)
    acc1 = _msg_kernel(hs1, edata, ewd)
    hs2 = _tc_stage2(acc1, hs1, dinv, b1r, W2)
    acc2 = _msg_kernel(hs2, edata, ewd)
    out = _tc_stage3(acc2, hs2, dinv, b2r)
    return out
